# baseline (speedup 1.0000x reference)
import functools

import jax
import jax.numpy as jnp
from jax import lax
from jax.experimental import pallas as pl
from jax.experimental.pallas import tpu as pltpu
from jax.experimental.pallas import tpu_sc as plsc

NC = 2
NS = 16
NW = NC * NS
CHUNK = 128


def _mesh():
  return plsc.VectorSubcoreMesh(core_axis_name="c", subcore_axis_name="s")


def _make_deg(e_pad, hb, n):
  per_tile = e_pad // NW

  @functools.partial(
      pl.kernel,
      out_type=jax.ShapeDtypeStruct((NC, hb, 128), jnp.float32),
      mesh=_mesh(),
      compiler_params=pltpu.CompilerParams(needs_layout_passes=False),
      scratch_types=[
          pltpu.VMEM((per_tile,), jnp.int32),
          pltpu.VMEM((hb, 128), jnp.float32),
          pltpu.VMEM((hb,), jnp.int32),
          pltpu.VMEM_SHARED((hb, 128), jnp.float32),
      ],
  )
  def deg_kernel(rowp, zeros, out, rowv, hist, iotav, shist):
    c = lax.axis_index("c")
    s = lax.axis_index("s")
    wid = c * NS + s

    @pl.when(s == 0)
    def _():
      pltpu.sync_copy(zeros.at[pl.ds(0, hb)], shist)

    def zero_row(r, _):
      def zero_col(j, _):
        hist[r, pl.ds(j * 16, 16)] = jnp.zeros((16,), jnp.float32)
        return ()
      lax.fori_loop(0, 8, zero_col, ())
      return ()
    lax.fori_loop(0, hb, zero_row, ())

    def fill_iota(i, _):
      iotav[pl.ds(i * 16, 16)] = lax.iota(jnp.int32, 16) + i * 16
      return ()
    lax.fori_loop(0, hb // 16, fill_iota, ())

    pltpu.sync_copy(rowp.at[pl.ds(wid * per_tile, per_tile)], rowv)

    ones = jnp.ones((16,), jnp.float32)

    def body(i, _):
      idx = rowv[pl.ds(i * 16, 16)]
      ir = lax.shift_right_logical(idx, 7)
      ic = lax.bitwise_and(idx, 127)
      plsc.addupdate_scatter(hist, [ir, ic], ones)
      return ()

    lax.fori_loop(0, per_tile // 16, body, ())
    plsc.subcore_barrier()

    pltpu.sync_copy(hist, shist.at[iotav], add=True)
    plsc.subcore_barrier()

    @pl.when(s == 0)
    def _():
      pltpu.sync_copy(shist, out.at[c])

  return deg_kernel


def _make_agg(e_pad, n_acc, n):
  chunks = e_pad // (NW * CHUNK)
  assert chunks % 2 == 0
  ngroups = chunks // 2

  @functools.partial(
      pl.kernel,
      out_type=jax.ShapeDtypeStruct((NC, n, 128), jnp.float32),
      mesh=_mesh(),
      scratch_types=[
          pltpu.VMEM((CHUNK,), jnp.int32),
          pltpu.VMEM((CHUNK,), jnp.int32),
          pltpu.VMEM((CHUNK,), jnp.int32),
          pltpu.VMEM((CHUNK,), jnp.int32),
          pltpu.VMEM((CHUNK, 128), jnp.float32),
          pltpu.VMEM((CHUNK, 128), jnp.float32),
          pltpu.VMEM_SHARED((n_acc, 128), jnp.float32),
          pltpu.SemaphoreType.DMA,
          pltpu.SemaphoreType.DMA,
      ],
  )
  def agg_kernel(src, zeros, colp, rowp, out, colA, rowA, colB, rowB,
                 gbufA, gbufB, acc, gsemA, gsemB):
    c = lax.axis_index("c")
    s = lax.axis_index("s")
    wid = c * NS + s

    @pl.when((s == 0) & (c == 0))
    def _():
      pltpu.sync_copy(src, acc.at[pl.ds(0, n)])

    @pl.when((s == 0) & (c == 1))
    def _():
      pltpu.sync_copy(zeros, acc.at[pl.ds(0, n)])

    def load_idx(ci, colv, rowv):
      base = (wid * chunks + ci) * CHUNK
      pltpu.sync_copy(colp.at[pl.ds(base, CHUNK)], colv)
      pltpu.sync_copy(rowp.at[pl.ds(base, CHUNK)], rowv)

    def g_desc(colv, gbuf, sem):
      return pltpu.make_async_copy(src.at[colv], gbuf, sem)

    plsc.subcore_barrier()

    def body(i, _):
      load_idx(i, colA, rowA)
      g_desc(colA, gbufA, gsemA).start()
      g_desc(colA, gbufA, gsemA).wait()
      pltpu.sync_copy(gbufA, acc.at[rowA], add=True)
      return ()

    lax.fori_loop(0, chunks, body, (), unroll=False)
    plsc.subcore_barrier()

    @pl.when(s == 0)
    def _():
      pltpu.sync_copy(acc.at[pl.ds(0, n)], out.at[c])

  return agg_kernel


def _tc0_body(p_ref, out_ref):
  d = p_ref[0] + p_ref[1] + 1.0
  out_ref[...] = jnp.where(d > 0.0, lax.rsqrt(d), 0.0)


def _tc1_body(x_ref, w1t_ref, b1_ref, dis_ref, out_ref):
  h = jnp.dot(x_ref[...], w1t_ref[...], preferred_element_type=jnp.float32)
  out_ref[...] = (h + b1_ref[...]) * dis_ref[...]


def _tc2_body(p_ref, dis_ref, w2t_ref, b2_ref, out_ref):
  dis = dis_ref[...]
  a = (p_ref[0] + p_ref[1]) * dis
  r = jnp.maximum(a, 0.0)
  h = jnp.dot(r, w2t_ref[...], preferred_element_type=jnp.float32)
  out_ref[...] = (h + b2_ref[...]) * dis


def _tc3_body(p_ref, dis_ref, out_ref):
  out_ref[...] = (p_ref[0] + p_ref[1]) * dis_ref[...]


def kernel(x, edge_index, num_nodes, W1, b1, W2, b2):
  n, d_in = x.shape
  e = edge_index.shape[1]
  del num_nodes

  chunks_pt = -(-e // (NW * CHUNK))
  chunks_pt = -(-chunks_pt // 2) * 2
  per_tile = chunks_pt * CHUNK
  e_pad = per_tile * NW
  n_acc = n + 8
  hb = -(-(n + 1) // 128)
  hb = -(-hb // 16) * 16

  row = edge_index[0]
  col = edge_index[1]
  pad = e_pad - e
  rowp = jnp.concatenate([row, jnp.full((pad,), n, jnp.int32)])
  colp = jnp.concatenate([col, jnp.zeros((pad,), jnp.int32)])
  rowp2 = rowp.reshape(-1, CHUNK)
  colp2 = colp.reshape(-1, CHUNK)

  zeros = jnp.zeros((n, 128), jnp.float32)

  degp = _make_deg(e_pad, hb, n)(rowp, zeros)

  dis2d = pl.pallas_call(
      _tc0_body,
      grid=(1,),
      in_specs=[pl.BlockSpec((NC, hb, 128), lambda i: (0, 0, 0))],
      out_specs=pl.BlockSpec((hb, 128), lambda i: (0, 0)),
      out_shape=jax.ShapeDtypeStruct((hb, 128), jnp.float32),
  )(degp)
  dis = dis2d.reshape(-1)[:n, None]

  grid = 10
  blk = n // grid
  dis_spec = pl.BlockSpec((blk, 1), lambda i: (i, 0))
  mat_spec = pl.BlockSpec((d_in, 128), lambda i: (0, 0))
  bias_spec = pl.BlockSpec((1, 128), lambda i: (0, 0))
  row_spec = pl.BlockSpec((blk, 128), lambda i: (i, 0))
  part_spec = pl.BlockSpec((NC, blk, 128), lambda i: (0, i, 0))

  s1 = pl.pallas_call(
      _tc1_body,
      grid=(grid,),
      in_specs=[row_spec, mat_spec, bias_spec, dis_spec],
      out_specs=row_spec,
      out_shape=jax.ShapeDtypeStruct((n, 128), jnp.float32),
  )(x, W1.T, b1[None, :], dis)

  agg = _make_agg(e_pad, n_acc, n)
  p1 = agg(s1, zeros, colp, rowp)

  s2 = pl.pallas_call(
      _tc2_body,
      grid=(grid,),
      in_specs=[part_spec, dis_spec, mat_spec, bias_spec],
      out_specs=row_spec,
      out_shape=jax.ShapeDtypeStruct((n, 128), jnp.float32),
  )(p1, dis, W2.T, b2[None, :])

  p2 = agg(s2, zeros, colp, rowp)

  out = pl.pallas_call(
      _tc3_body,
      grid=(grid,),
      in_specs=[part_spec, dis_spec],
      out_specs=row_spec,
      out_shape=jax.ShapeDtypeStruct((n, 128), jnp.float32),
  )(p2, dis)

  return out


# speedup vs baseline: 1.0011x; 1.0011x over previous
import functools

import jax
import jax.numpy as jnp
from jax import lax
from jax.experimental import pallas as pl
from jax.experimental.pallas import tpu as pltpu
from jax.experimental.pallas import tpu_sc as plsc

NC = 2
NS = 16
NW = NC * NS
CHUNK = 128


def _mesh():
  return plsc.VectorSubcoreMesh(core_axis_name="c", subcore_axis_name="s")


def _make_deg(e_pad, hb, n):
  per_tile = e_pad // NW

  @functools.partial(
      pl.kernel,
      out_type=jax.ShapeDtypeStruct((NC, hb, 128), jnp.float32),
      mesh=_mesh(),
      compiler_params=pltpu.CompilerParams(needs_layout_passes=False),
      scratch_types=[
          pltpu.VMEM((per_tile,), jnp.int32),
          pltpu.VMEM((hb, 128), jnp.float32),
          pltpu.VMEM((hb,), jnp.int32),
          pltpu.VMEM_SHARED((hb, 128), jnp.float32),
      ],
  )
  def deg_kernel(rowp, zeros, out, rowv, hist, iotav, shist):
    c = lax.axis_index("c")
    s = lax.axis_index("s")
    wid = c * NS + s

    @pl.when(s == 0)
    def _():
      pltpu.sync_copy(zeros.at[pl.ds(0, hb)], shist)

    def zero_row(r, _):
      def zero_col(j, _):
        hist[r, pl.ds(j * 16, 16)] = jnp.zeros((16,), jnp.float32)
        return ()
      lax.fori_loop(0, 8, zero_col, ())
      return ()
    lax.fori_loop(0, hb, zero_row, ())

    def fill_iota(i, _):
      iotav[pl.ds(i * 16, 16)] = lax.iota(jnp.int32, 16) + i * 16
      return ()
    lax.fori_loop(0, hb // 16, fill_iota, ())

    pltpu.sync_copy(rowp.at[pl.ds(wid * per_tile, per_tile)], rowv)

    ones = jnp.ones((16,), jnp.float32)

    def body(i, _):
      idx = rowv[pl.ds(i * 16, 16)]
      ir = lax.shift_right_logical(idx, 7)
      ic = lax.bitwise_and(idx, 127)
      plsc.addupdate_scatter(hist, [ir, ic], ones)
      return ()

    lax.fori_loop(0, per_tile // 16, body, ())
    plsc.subcore_barrier()

    pltpu.sync_copy(hist, shist.at[iotav], add=True)
    plsc.subcore_barrier()

    @pl.when(s == 0)
    def _():
      pltpu.sync_copy(shist, out.at[c])

  return deg_kernel


def _make_agg(e_pad, n_acc, n):
  chunks = e_pad // (NW * CHUNK)
  assert chunks % 2 == 0
  ngroups = chunks // 2

  @functools.partial(
      pl.kernel,
      out_type=jax.ShapeDtypeStruct((NC, n, 128), jnp.float32),
      mesh=_mesh(),
      scratch_types=[
          pltpu.VMEM((CHUNK,), jnp.int32),
          pltpu.VMEM((CHUNK,), jnp.int32),
          pltpu.VMEM((CHUNK,), jnp.int32),
          pltpu.VMEM((CHUNK,), jnp.int32),
          pltpu.VMEM((CHUNK, 128), jnp.float32),
          pltpu.VMEM((CHUNK, 128), jnp.float32),
          pltpu.VMEM_SHARED((n_acc, 128), jnp.float32),
          pltpu.SemaphoreType.DMA,
          pltpu.SemaphoreType.DMA,
      ],
  )
  def agg_kernel(src, zeros, colp, rowp, out, colA, rowA, colB, rowB,
                 gbufA, gbufB, acc, gsemA, gsemB):
    c = lax.axis_index("c")
    s = lax.axis_index("s")
    wid = c * NS + s

    @pl.when((s == 0) & (c == 0))
    def _():
      pltpu.sync_copy(src, acc.at[pl.ds(0, n)])

    @pl.when((s == 0) & (c == 1))
    def _():
      pltpu.sync_copy(zeros, acc.at[pl.ds(0, n)])

    def load_idx(ci, colv, rowv):
      base = (wid * chunks + ci) * CHUNK
      pltpu.sync_copy(colp.at[pl.ds(base, CHUNK)], colv)
      pltpu.sync_copy(rowp.at[pl.ds(base, CHUNK)], rowv)

    def g_desc(colv, gbuf, sem):
      return pltpu.make_async_copy(src.at[colv], gbuf, sem)

    plsc.subcore_barrier()

    def body(i, _):
      load_idx(i, colA, rowA)
      g_desc(colA, gbufA, gsemA).start()
      g_desc(colA, gbufA, gsemA).wait()
      pltpu.sync_copy(gbufA, acc.at[plsc.Indices(rowA, ignored_value=n)],
                      add=True)
      return ()

    lax.fori_loop(0, chunks, body, (), unroll=False)
    plsc.subcore_barrier()

    @pl.when(s == 0)
    def _():
      pltpu.sync_copy(acc.at[pl.ds(0, n)], out.at[c])

  return agg_kernel


def _tc0_body(p_ref, out_ref):
  d = p_ref[0] + p_ref[1] + 1.0
  out_ref[...] = jnp.where(d > 0.0, lax.rsqrt(d), 0.0)


def _tc1_body(x_ref, w1t_ref, b1_ref, dis_ref, out_ref):
  h = jnp.dot(x_ref[...], w1t_ref[...], preferred_element_type=jnp.float32)
  out_ref[...] = (h + b1_ref[...]) * dis_ref[...]


def _tc2_body(p_ref, dis_ref, w2t_ref, b2_ref, out_ref):
  dis = dis_ref[...]
  a = (p_ref[0] + p_ref[1]) * dis
  r = jnp.maximum(a, 0.0)
  h = jnp.dot(r, w2t_ref[...], preferred_element_type=jnp.float32)
  out_ref[...] = (h + b2_ref[...]) * dis


def _tc3_body(p_ref, dis_ref, out_ref):
  out_ref[...] = (p_ref[0] + p_ref[1]) * dis_ref[...]


def kernel(x, edge_index, num_nodes, W1, b1, W2, b2):
  n, d_in = x.shape
  e = edge_index.shape[1]
  del num_nodes

  chunks_pt = -(-e // (NW * CHUNK))
  chunks_pt = -(-chunks_pt // 2) * 2
  per_tile = chunks_pt * CHUNK
  e_pad = per_tile * NW
  n_acc = n + 8
  hb = -(-(n + 1) // 128)
  hb = -(-hb // 16) * 16

  row = edge_index[0]
  col = edge_index[1]
  pad = e_pad - e
  rowp = jnp.concatenate([row, jnp.full((pad,), n, jnp.int32)])
  colp = jnp.concatenate([col, jnp.zeros((pad,), jnp.int32)])
  rowp2 = rowp.reshape(-1, CHUNK)
  colp2 = colp.reshape(-1, CHUNK)

  zeros = jnp.zeros((n, 128), jnp.float32)

  degp = _make_deg(e_pad, hb, n)(rowp, zeros)

  dis2d = pl.pallas_call(
      _tc0_body,
      grid=(1,),
      in_specs=[pl.BlockSpec((NC, hb, 128), lambda i: (0, 0, 0))],
      out_specs=pl.BlockSpec((hb, 128), lambda i: (0, 0)),
      out_shape=jax.ShapeDtypeStruct((hb, 128), jnp.float32),
  )(degp)
  dis = dis2d.reshape(-1)[:n, None]

  grid = 10
  blk = n // grid
  dis_spec = pl.BlockSpec((blk, 1), lambda i: (i, 0))
  mat_spec = pl.BlockSpec((d_in, 128), lambda i: (0, 0))
  bias_spec = pl.BlockSpec((1, 128), lambda i: (0, 0))
  row_spec = pl.BlockSpec((blk, 128), lambda i: (i, 0))
  part_spec = pl.BlockSpec((NC, blk, 128), lambda i: (0, i, 0))

  s1 = pl.pallas_call(
      _tc1_body,
      grid=(grid,),
      in_specs=[row_spec, mat_spec, bias_spec, dis_spec],
      out_specs=row_spec,
      out_shape=jax.ShapeDtypeStruct((n, 128), jnp.float32),
  )(x, W1.T, b1[None, :], dis)

  agg = _make_agg(e_pad, n_acc, n)
  p1 = agg(s1, zeros, colp, rowp)

  s2 = pl.pallas_call(
      _tc2_body,
      grid=(grid,),
      in_specs=[part_spec, dis_spec, mat_spec, bias_spec],
      out_specs=row_spec,
      out_shape=jax.ShapeDtypeStruct((n, 128), jnp.float32),
  )(p1, dis, W2.T, b2[None, :])

  p2 = agg(s2, zeros, colp, rowp)

  out = pl.pallas_call(
      _tc3_body,
      grid=(grid,),
      in_specs=[part_spec, dis_spec],
      out_specs=row_spec,
      out_shape=jax.ShapeDtypeStruct((n, 128), jnp.float32),
  )(p2, dis)

  return out


# speedup vs baseline: 1.7275x; 1.7256x over previous
import functools

import jax
import jax.numpy as jnp
from jax import lax
from jax.experimental import pallas as pl
from jax.experimental.pallas import tpu as pltpu
from jax.experimental.pallas import tpu_sc as plsc

NC = 2
NS = 16
NW = NC * NS
CHUNK = 128


def _mesh():
  return plsc.VectorSubcoreMesh(core_axis_name="c", subcore_axis_name="s")


def _make_deg(e_pad, hb, n):
  per_tile = e_pad // NW

  @functools.partial(
      pl.kernel,
      out_type=jax.ShapeDtypeStruct((NC, hb, 128), jnp.float32),
      mesh=_mesh(),
      compiler_params=pltpu.CompilerParams(needs_layout_passes=False),
      scratch_types=[
          pltpu.VMEM((per_tile,), jnp.int32),
          pltpu.VMEM((hb, 128), jnp.float32),
          pltpu.VMEM((hb,), jnp.int32),
          pltpu.VMEM_SHARED((hb, 128), jnp.float32),
      ],
  )
  def deg_kernel(rowp, zeros, out, rowv, hist, iotav, shist):
    c = lax.axis_index("c")
    s = lax.axis_index("s")
    wid = c * NS + s

    @pl.when(s == 0)
    def _():
      pltpu.sync_copy(zeros.at[pl.ds(0, hb)], shist)

    def zero_row(r, _):
      def zero_col(j, _):
        hist[r, pl.ds(j * 16, 16)] = jnp.zeros((16,), jnp.float32)
        return ()
      lax.fori_loop(0, 8, zero_col, ())
      return ()
    lax.fori_loop(0, hb, zero_row, ())

    def fill_iota(i, _):
      iotav[pl.ds(i * 16, 16)] = lax.iota(jnp.int32, 16) + i * 16
      return ()
    lax.fori_loop(0, hb // 16, fill_iota, ())

    pltpu.sync_copy(rowp.at[pl.ds(wid * per_tile, per_tile)], rowv)

    ones = jnp.ones((16,), jnp.float32)

    def body(i, _):
      idx = rowv[pl.ds(i * 16, 16)]
      ir = lax.shift_right_logical(idx, 7)
      ic = lax.bitwise_and(idx, 127)
      plsc.addupdate_scatter(hist, [ir, ic], ones)
      return ()

    lax.fori_loop(0, per_tile // 16, body, ())
    plsc.subcore_barrier()

    pltpu.sync_copy(hist, shist.at[iotav], add=True)
    plsc.subcore_barrier()

    @pl.when(s == 0)
    def _():
      pltpu.sync_copy(shist, out.at[c])

  return deg_kernel


def _make_agg(e_pad, n_acc, n):
  chunks = e_pad // (NW * CHUNK)

  @functools.partial(
      pl.kernel,
      out_type=jax.ShapeDtypeStruct((NC, n, 128), jnp.float32),
      mesh=_mesh(),
      scratch_types=[
          pltpu.VMEM((CHUNK,), jnp.int32),
          pltpu.VMEM((CHUNK,), jnp.int32),
          pltpu.VMEM((CHUNK, 128), jnp.float32),
          pltpu.VMEM_SHARED((n_acc, 128), jnp.float32),
          pltpu.SemaphoreType.DMA,
      ],
  )
  def agg_kernel(src, zeros, colp, rowp, out, colv, rowv, gbuf, acc, sem):
    c = lax.axis_index("c")
    s = lax.axis_index("s")
    wid = c * NS + s

    @pl.when((s == 0) & (c == 0))
    def _():
      pltpu.sync_copy(src, acc.at[pl.ds(0, n)])

    @pl.when((s == 0) & (c == 1))
    def _():
      pltpu.sync_copy(zeros, acc.at[pl.ds(0, n)])

    plsc.subcore_barrier()

    def body(i, _):
      base = (wid * chunks + i) * CHUNK
      pltpu.sync_copy(colp.at[pl.ds(base, CHUNK)], colv)
      pltpu.sync_copy(rowp.at[pl.ds(base, CHUNK)], rowv)
      pltpu.async_copy(src.at[colv], gbuf, sem).wait()
      pltpu.sync_copy(gbuf, acc.at[rowv], add=True)
      return ()

    lax.fori_loop(0, chunks, body, (), unroll=False)
    plsc.subcore_barrier()

    @pl.when(s == 0)
    def _():
      pltpu.sync_copy(acc.at[pl.ds(0, n)], out.at[c])

  return agg_kernel


def _tc0_body(p_ref, out_ref):
  d = p_ref[0] + p_ref[1] + 1.0
  out_ref[...] = jnp.where(d > 0.0, lax.rsqrt(d), 0.0)


def _tc1_body(x_ref, w1t_ref, b1_ref, dis_ref, out_ref):
  h = jnp.dot(x_ref[...], w1t_ref[...], preferred_element_type=jnp.float32)
  out_ref[...] = (h + b1_ref[...]) * dis_ref[...]


def _tc2_body(p_ref, dis_ref, w2t_ref, b2_ref, out_ref):
  dis = dis_ref[...]
  a = (p_ref[0] + p_ref[1]) * dis
  r = jnp.maximum(a, 0.0)
  h = jnp.dot(r, w2t_ref[...], preferred_element_type=jnp.float32)
  out_ref[...] = (h + b2_ref[...]) * dis


def _tc3_body(p_ref, dis_ref, out_ref):
  out_ref[...] = (p_ref[0] + p_ref[1]) * dis_ref[...]


def kernel(x, edge_index, num_nodes, W1, b1, W2, b2):
  n, d_in = x.shape
  e = edge_index.shape[1]
  del num_nodes

  chunks_pt = -(-e // (NW * CHUNK))
  per_tile = chunks_pt * CHUNK
  e_pad = per_tile * NW
  n_acc = n + 8
  hb = -(-(n + 1) // 128)
  hb = -(-hb // 16) * 16

  row = edge_index[0]
  col = edge_index[1]
  pad = e_pad - e
  rowp = jnp.concatenate([row, jnp.full((pad,), n, jnp.int32)])
  colp = jnp.concatenate([col, jnp.zeros((pad,), jnp.int32)])
  rowp2 = rowp.reshape(-1, CHUNK)
  colp2 = colp.reshape(-1, CHUNK)

  zeros = jnp.zeros((n, 128), jnp.float32)

  degp = _make_deg(e_pad, hb, n)(rowp, zeros)

  dis2d = pl.pallas_call(
      _tc0_body,
      grid=(1,),
      in_specs=[pl.BlockSpec((NC, hb, 128), lambda i: (0, 0, 0))],
      out_specs=pl.BlockSpec((hb, 128), lambda i: (0, 0)),
      out_shape=jax.ShapeDtypeStruct((hb, 128), jnp.float32),
  )(degp)
  dis = dis2d.reshape(-1)[:n, None]

  grid = 10
  blk = n // grid
  dis_spec = pl.BlockSpec((blk, 1), lambda i: (i, 0))
  mat_spec = pl.BlockSpec((d_in, 128), lambda i: (0, 0))
  bias_spec = pl.BlockSpec((1, 128), lambda i: (0, 0))
  row_spec = pl.BlockSpec((blk, 128), lambda i: (i, 0))
  part_spec = pl.BlockSpec((NC, blk, 128), lambda i: (0, i, 0))

  s1 = pl.pallas_call(
      _tc1_body,
      grid=(grid,),
      in_specs=[row_spec, mat_spec, bias_spec, dis_spec],
      out_specs=row_spec,
      out_shape=jax.ShapeDtypeStruct((n, 128), jnp.float32),
  )(x, W1.T, b1[None, :], dis)

  agg = _make_agg(e_pad, n_acc, n)
  p1 = agg(s1, zeros, colp, rowp)

  s2 = pl.pallas_call(
      _tc2_body,
      grid=(grid,),
      in_specs=[part_spec, dis_spec, mat_spec, bias_spec],
      out_specs=row_spec,
      out_shape=jax.ShapeDtypeStruct((n, 128), jnp.float32),
  )(p1, dis, W2.T, b2[None, :])

  p2 = agg(s2, zeros, colp, rowp)

  out = pl.pallas_call(
      _tc3_body,
      grid=(grid,),
      in_specs=[part_spec, dis_spec],
      out_specs=row_spec,
      out_shape=jax.ShapeDtypeStruct((n, 128), jnp.float32),
  )(p2, dis)

  return out


# speedup vs baseline: 2.5059x; 1.4506x over previous
import functools

import jax
import jax.numpy as jnp
from jax import lax
from jax.experimental import pallas as pl
from jax.experimental.pallas import tpu as pltpu
from jax.experimental.pallas import tpu_sc as plsc

NC = 2
NS = 16
NW = NC * NS
CHUNK = 128


def _mesh():
  return plsc.VectorSubcoreMesh(core_axis_name="c", subcore_axis_name="s")


def _make_deg(e_pad, hb, n):
  per_tile = e_pad // NW

  @functools.partial(
      pl.kernel,
      out_type=jax.ShapeDtypeStruct((NC, hb, 128), jnp.float32),
      mesh=_mesh(),
      compiler_params=pltpu.CompilerParams(needs_layout_passes=False),
      scratch_types=[
          pltpu.VMEM((per_tile,), jnp.int32),
          pltpu.VMEM((hb, 128), jnp.float32),
          pltpu.VMEM((hb,), jnp.int32),
          pltpu.VMEM_SHARED((hb, 128), jnp.float32),
      ],
  )
  def deg_kernel(rowp, zeros, out, rowv, hist, iotav, shist):
    c = lax.axis_index("c")
    s = lax.axis_index("s")
    wid = c * NS + s

    @pl.when(s == 0)
    def _():
      pltpu.sync_copy(zeros.at[pl.ds(0, hb)], shist)

    def zero_row(r, _):
      def zero_col(j, _):
        hist[r, pl.ds(j * 16, 16)] = jnp.zeros((16,), jnp.float32)
        return ()
      lax.fori_loop(0, 8, zero_col, ())
      return ()
    lax.fori_loop(0, hb, zero_row, ())

    def fill_iota(i, _):
      iotav[pl.ds(i * 16, 16)] = lax.iota(jnp.int32, 16) + i * 16
      return ()
    lax.fori_loop(0, hb // 16, fill_iota, ())

    pltpu.sync_copy(rowp.at[pl.ds(wid * per_tile, per_tile)], rowv)

    ones = jnp.ones((16,), jnp.float32)

    def body(i, _):
      idx = rowv[pl.ds(i * 16, 16)]
      ir = lax.shift_right_logical(idx, 7)
      ic = lax.bitwise_and(idx, 127)
      plsc.addupdate_scatter(hist, [ir, ic], ones)
      return ()

    lax.fori_loop(0, per_tile // 16, body, ())
    plsc.subcore_barrier()

    pltpu.sync_copy(hist, shist.at[iotav], add=True)
    plsc.subcore_barrier()

    @pl.when(s == 0)
    def _():
      pltpu.sync_copy(shist, out.at[c])

  return deg_kernel


def _make_agg(e_pad, n_acc, n):
  chunks = e_pad // (NW * CHUNK)

  @functools.partial(
      pl.kernel,
      out_type=jax.ShapeDtypeStruct((NC, n, 128), jnp.float32),
      mesh=_mesh(),
      scratch_types=[
          pltpu.VMEM((CHUNK,), jnp.int32),
          pltpu.VMEM((CHUNK,), jnp.int32),
          pltpu.VMEM((CHUNK, 128), jnp.float32),
          pltpu.VMEM_SHARED((n_acc, 128), jnp.float32),
          pltpu.SemaphoreType.DMA,
      ],
  )
  def agg_kernel(src, zeros, colp, rowp, out, colv, rowv, gbuf, acc, sem):
    c = lax.axis_index("c")
    s = lax.axis_index("s")
    wid = c * NS + s

    @pl.when((s == 0) & (c == 0))
    def _():
      pltpu.sync_copy(src, acc.at[pl.ds(0, n)])

    @pl.when((s == 0) & (c == 1))
    def _():
      pltpu.sync_copy(zeros, acc.at[pl.ds(0, n)])

    plsc.subcore_barrier()

    def body(i, _):
      base = (wid * chunks + i) * CHUNK
      pltpu.sync_copy(colp.at[pl.ds(base, CHUNK)], colv)
      pltpu.sync_copy(rowp.at[pl.ds(base, CHUNK)], rowv)
      pltpu.async_copy(src.at[colv], gbuf, sem).wait()
      pltpu.sync_copy(gbuf, acc.at[rowv], add=True)
      return ()

    lax.fori_loop(0, chunks, body, (), unroll=False)
    plsc.subcore_barrier()

    @pl.when(s == 0)
    def _():
      pltpu.sync_copy(acc.at[pl.ds(0, n)], out.at[c])

  return agg_kernel


def _tc0_body(p_ref, out_ref):
  d = p_ref[0] + p_ref[1] + 1.0
  out_ref[...] = jnp.where(d > 0.0, lax.rsqrt(d), 0.0)


def _tc1_body(x_ref, w1t_ref, b1_ref, dis_ref, out_ref):
  h = jnp.dot(x_ref[...], w1t_ref[...], preferred_element_type=jnp.float32)
  out_ref[...] = (h + b1_ref[...]) * dis_ref[...]


def _tc2_body(p_ref, dis_ref, w2t_ref, b2_ref, out_ref):
  dis = dis_ref[...]
  a = (p_ref[0] + p_ref[1]) * dis
  r = jnp.maximum(a, 0.0)
  h = jnp.dot(r, w2t_ref[...], preferred_element_type=jnp.float32)
  out_ref[...] = (h + b2_ref[...]) * dis


def _tc3_body(p_ref, dis_ref, out_ref):
  out_ref[...] = (p_ref[0] + p_ref[1]) * dis_ref[...]


def kernel(x, edge_index, num_nodes, W1, b1, W2, b2):
  n, d_in = x.shape
  e = edge_index.shape[1]
  del num_nodes

  chunks_pt = -(-e // (NW * CHUNK))
  chunks_pt = -(-chunks_pt // 2) * 2
  per_tile = chunks_pt * CHUNK
  e_pad = per_tile * NW
  n_acc = n + 2048
  hb = -(-n_acc // 128)
  hb = -(-hb // 16) * 16

  row = edge_index[0]
  col = edge_index[1]
  pad = e_pad - e
  pad_idx = jnp.arange(pad, dtype=jnp.int32)
  rowp = jnp.concatenate([row, n + (pad_idx % 2048)])
  colp = jnp.concatenate([col, pad_idx % n])
  rowp2 = rowp.reshape(-1, CHUNK)
  colp2 = colp.reshape(-1, CHUNK)

  zeros = jnp.zeros((n, 128), jnp.float32)

  degp = _make_deg(e_pad, hb, n)(rowp, zeros)

  dis2d = pl.pallas_call(
      _tc0_body,
      grid=(1,),
      in_specs=[pl.BlockSpec((NC, hb, 128), lambda i: (0, 0, 0))],
      out_specs=pl.BlockSpec((hb, 128), lambda i: (0, 0)),
      out_shape=jax.ShapeDtypeStruct((hb, 128), jnp.float32),
  )(degp)
  dis = dis2d.reshape(-1)[:n, None]

  grid = 10
  blk = n // grid
  dis_spec = pl.BlockSpec((blk, 1), lambda i: (i, 0))
  mat_spec = pl.BlockSpec((d_in, 128), lambda i: (0, 0))
  bias_spec = pl.BlockSpec((1, 128), lambda i: (0, 0))
  row_spec = pl.BlockSpec((blk, 128), lambda i: (i, 0))
  part_spec = pl.BlockSpec((NC, blk, 128), lambda i: (0, i, 0))

  s1 = pl.pallas_call(
      _tc1_body,
      grid=(grid,),
      in_specs=[row_spec, mat_spec, bias_spec, dis_spec],
      out_specs=row_spec,
      out_shape=jax.ShapeDtypeStruct((n, 128), jnp.float32),
  )(x, W1.T, b1[None, :], dis)

  agg = _make_agg(e_pad, n_acc, n)
  p1 = agg(s1, zeros, colp, rowp)

  s2 = pl.pallas_call(
      _tc2_body,
      grid=(grid,),
      in_specs=[part_spec, dis_spec, mat_spec, bias_spec],
      out_specs=row_spec,
      out_shape=jax.ShapeDtypeStruct((n, 128), jnp.float32),
  )(p1, dis, W2.T, b2[None, :])

  p2 = agg(s2, zeros, colp, rowp)

  out = pl.pallas_call(
      _tc3_body,
      grid=(grid,),
      in_specs=[part_spec, dis_spec],
      out_specs=row_spec,
      out_shape=jax.ShapeDtypeStruct((n, 128), jnp.float32),
  )(p2, dis)

  return out


# speedup vs baseline: 3.7924x; 1.5134x over previous
import functools

import jax
import jax.numpy as jnp
from jax import lax
from jax.experimental import pallas as pl
from jax.experimental.pallas import tpu as pltpu
from jax.experimental.pallas import tpu_sc as plsc

NC = 2
NS = 16
NW = NC * NS
CHUNK = 128


def _mesh():
  return plsc.VectorSubcoreMesh(core_axis_name="c", subcore_axis_name="s")


def _make_deg(e_pad, hb, n):
  per_tile = e_pad // NW

  @functools.partial(
      pl.kernel,
      out_type=jax.ShapeDtypeStruct((NC, hb, 128), jnp.float32),
      mesh=_mesh(),
      compiler_params=pltpu.CompilerParams(needs_layout_passes=False),
      scratch_types=[
          pltpu.VMEM((per_tile,), jnp.int32),
          pltpu.VMEM((hb, 128), jnp.float32),
          pltpu.VMEM((hb,), jnp.int32),
          pltpu.VMEM_SHARED((hb, 128), jnp.float32),
      ],
  )
  def deg_kernel(rowp, zeros, out, rowv, hist, iotav, shist):
    c = lax.axis_index("c")
    s = lax.axis_index("s")
    wid = c * NS + s

    @pl.when(s == 0)
    def _():
      pltpu.sync_copy(zeros.at[pl.ds(0, hb)], shist)

    def zero_row(r, _):
      def zero_col(j, _):
        hist[r, pl.ds(j * 16, 16)] = jnp.zeros((16,), jnp.float32)
        return ()
      lax.fori_loop(0, 8, zero_col, ())
      return ()
    lax.fori_loop(0, hb, zero_row, ())

    def fill_iota(i, _):
      iotav[pl.ds(i * 16, 16)] = lax.iota(jnp.int32, 16) + i * 16
      return ()
    lax.fori_loop(0, hb // 16, fill_iota, ())

    pltpu.sync_copy(rowp.at[pl.ds(wid * per_tile, per_tile)], rowv)

    ones = jnp.ones((16,), jnp.float32)

    def body(i, _):
      idx = rowv[pl.ds(i * 16, 16)]
      ir = lax.shift_right_logical(idx, 7)
      ic = lax.bitwise_and(idx, 127)
      plsc.addupdate_scatter(hist, [ir, ic], ones)
      return ()

    lax.fori_loop(0, per_tile // 16, body, ())
    plsc.subcore_barrier()

    pltpu.sync_copy(hist, shist.at[iotav], add=True)
    plsc.subcore_barrier()

    @pl.when(s == 0)
    def _():
      pltpu.sync_copy(shist, out.at[c])

  return deg_kernel


def _make_agg(e_pad, n_acc, n):
  chunks = e_pad // (NW * CHUNK)
  assert chunks % 2 == 0
  ngroups = chunks // 2

  @functools.partial(
      pl.kernel,
      out_type=jax.ShapeDtypeStruct((NC, n, 128), jnp.float32),
      mesh=_mesh(),
      scratch_types=[
          pltpu.VMEM((CHUNK,), jnp.int32),
          pltpu.VMEM((CHUNK,), jnp.int32),
          pltpu.VMEM((CHUNK,), jnp.int32),
          pltpu.VMEM((CHUNK,), jnp.int32),
          pltpu.VMEM((CHUNK, 128), jnp.float32),
          pltpu.VMEM((CHUNK, 128), jnp.float32),
          pltpu.VMEM_SHARED((n_acc, 128), jnp.float32),
          pltpu.SemaphoreType.DMA,
          pltpu.SemaphoreType.DMA,
      ],
  )
  def agg_kernel(src, zeros, colp, rowp, out, colA, rowA, colB, rowB,
                 gbufA, gbufB, acc, gsemA, gsemB):
    c = lax.axis_index("c")
    s = lax.axis_index("s")
    wid = c * NS + s

    @pl.when((s == 0) & (c == 0))
    def _():
      pltpu.sync_copy(src, acc.at[pl.ds(0, n)])

    @pl.when((s == 0) & (c == 1))
    def _():
      pltpu.sync_copy(zeros, acc.at[pl.ds(0, n)])

    def load_idx(ci, colv, rowv):
      base = (wid * chunks + ci) * CHUNK
      pltpu.sync_copy(colp.at[pl.ds(base, CHUNK)], colv)
      pltpu.sync_copy(rowp.at[pl.ds(base, CHUNK)], rowv)

    def g_desc(colv, gbuf, sem):
      return pltpu.make_async_copy(src.at[colv], gbuf, sem)

    plsc.subcore_barrier()
    load_idx(0, colA, rowA)
    g_desc(colA, gbufA, gsemA).start()

    def body(g, _):
      i0 = g * 2
      load_idx(i0 + 1, colB, rowB)
      g_desc(colA, gbufA, gsemA).wait()
      g_desc(colB, gbufB, gsemB).start()
      pltpu.sync_copy(gbufA, acc.at[rowA], add=True)

      @pl.when(g < ngroups - 1)
      def _():
        load_idx(i0 + 2, colA, rowA)

      g_desc(colB, gbufB, gsemB).wait()

      @pl.when(g < ngroups - 1)
      def _():
        g_desc(colA, gbufA, gsemA).start()

      pltpu.sync_copy(gbufB, acc.at[rowB], add=True)
      return ()

    lax.fori_loop(0, ngroups, body, (), unroll=False)
    plsc.subcore_barrier()

    @pl.when(s == 0)
    def _():
      pltpu.sync_copy(acc.at[pl.ds(0, n)], out.at[c])

  return agg_kernel


def _tc0_body(p_ref, out_ref):
  d = p_ref[0] + p_ref[1] + 1.0
  out_ref[...] = jnp.where(d > 0.0, lax.rsqrt(d), 0.0)


def _tc1_body(x_ref, w1t_ref, b1_ref, dis_ref, out_ref):
  h = jnp.dot(x_ref[...], w1t_ref[...], preferred_element_type=jnp.float32)
  out_ref[...] = (h + b1_ref[...]) * dis_ref[...]


def _tc2_body(p_ref, dis_ref, w2t_ref, b2_ref, out_ref):
  dis = dis_ref[...]
  a = (p_ref[0] + p_ref[1]) * dis
  r = jnp.maximum(a, 0.0)
  h = jnp.dot(r, w2t_ref[...], preferred_element_type=jnp.float32)
  out_ref[...] = (h + b2_ref[...]) * dis


def _tc3_body(p_ref, dis_ref, out_ref):
  out_ref[...] = (p_ref[0] + p_ref[1]) * dis_ref[...]


def kernel(x, edge_index, num_nodes, W1, b1, W2, b2):
  n, d_in = x.shape
  e = edge_index.shape[1]
  del num_nodes

  chunks_pt = -(-e // (NW * CHUNK))
  chunks_pt = -(-chunks_pt // 2) * 2
  per_tile = chunks_pt * CHUNK
  e_pad = per_tile * NW
  n_acc = n + 2048
  hb = -(-n_acc // 128)
  hb = -(-hb // 16) * 16

  row = edge_index[0]
  col = edge_index[1]
  pad = e_pad - e
  pad_idx = jnp.arange(pad, dtype=jnp.int32)
  rowp = jnp.concatenate([row, n + (pad_idx % 2048)])
  colp = jnp.concatenate([col, pad_idx % n])
  rowp2 = rowp.reshape(-1, CHUNK)
  colp2 = colp.reshape(-1, CHUNK)

  zeros = jnp.zeros((n, 128), jnp.float32)

  degp = _make_deg(e_pad, hb, n)(rowp, zeros)

  dis2d = pl.pallas_call(
      _tc0_body,
      grid=(1,),
      in_specs=[pl.BlockSpec((NC, hb, 128), lambda i: (0, 0, 0))],
      out_specs=pl.BlockSpec((hb, 128), lambda i: (0, 0)),
      out_shape=jax.ShapeDtypeStruct((hb, 128), jnp.float32),
  )(degp)
  dis = dis2d.reshape(-1)[:n, None]

  grid = 10
  blk = n // grid
  dis_spec = pl.BlockSpec((blk, 1), lambda i: (i, 0))
  mat_spec = pl.BlockSpec((d_in, 128), lambda i: (0, 0))
  bias_spec = pl.BlockSpec((1, 128), lambda i: (0, 0))
  row_spec = pl.BlockSpec((blk, 128), lambda i: (i, 0))
  part_spec = pl.BlockSpec((NC, blk, 128), lambda i: (0, i, 0))

  s1 = pl.pallas_call(
      _tc1_body,
      grid=(grid,),
      in_specs=[row_spec, mat_spec, bias_spec, dis_spec],
      out_specs=row_spec,
      out_shape=jax.ShapeDtypeStruct((n, 128), jnp.float32),
  )(x, W1.T, b1[None, :], dis)

  agg = _make_agg(e_pad, n_acc, n)
  p1 = agg(s1, zeros, colp, rowp)

  s2 = pl.pallas_call(
      _tc2_body,
      grid=(grid,),
      in_specs=[part_spec, dis_spec, mat_spec, bias_spec],
      out_specs=row_spec,
      out_shape=jax.ShapeDtypeStruct((n, 128), jnp.float32),
  )(p1, dis, W2.T, b2[None, :])

  p2 = agg(s2, zeros, colp, rowp)

  out = pl.pallas_call(
      _tc3_body,
      grid=(grid,),
      in_specs=[part_spec, dis_spec],
      out_specs=row_spec,
      out_shape=jax.ShapeDtypeStruct((n, 128), jnp.float32),
  )(p2, dis)

  return out
